# baseline (speedup 1.0000x reference)
import functools

import jax
import jax.numpy as jnp
from jax import lax
from jax.experimental import pallas as pl
from jax.experimental.pallas import tpu as pltpu
from jax.experimental.pallas import tpu_sc as plsc

_NUM_CORES = 2
_NUM_SUBCORES = 16
_NW = _NUM_CORES * _NUM_SUBCORES
_LANES = 16

_CHUNK = 64
_SUP_G = 5
_SUP = _SUP_G * _CHUNK


def _sc_gather_paired(embed, idx):
    M = idx.shape[0]
    F = embed.shape[1]
    H = M // 2
    per_w = H // _NW
    n_sup = per_w // _SUP
    mesh = plsc.VectorSubcoreMesh(core_axis_name="c", subcore_axis_name="s")

    @functools.partial(
        pl.kernel,
        mesh=mesh,
        out_type=jax.ShapeDtypeStruct((H, 2 * F), jnp.float32),
        compiler_params=pltpu.CompilerParams(use_tc_tiling_on_sc=False),
        scratch_types=[
            pltpu.VMEM((2, per_w), jnp.int32),
            pltpu.VMEM((2, _SUP, 2 * F), jnp.float32),
            pltpu.SemaphoreType.DMA,
            pltpu.SemaphoreType.DMA,
            pltpu.SemaphoreType.DMA,
            pltpu.SemaphoreType.DMA,
        ],
    )
    def gather_kernel(table_hbm, idx_hbm, out_hbm, idx_v, rows_v, gs0, gs1, ws0, ws1):
        wid = lax.axis_index("s") * _NUM_CORES + lax.axis_index("c")
        base = wid * per_w
        pltpu.sync_copy(idx_hbm.at[pl.ds(base, per_w)], idx_v.at[0])
        pltpu.sync_copy(idx_hbm.at[pl.ds(H + base, per_w)], idx_v.at[1])

        def clamp_body(i, carry):
            sl = pl.ds(i * _LANES, _LANES)
            idx_v[0, sl] = jnp.maximum(idx_v[0, sl], 1)
            idx_v[1, sl] = jnp.maximum(idx_v[1, sl], 1)
            return carry

        lax.fori_loop(0, per_w // _LANES, clamp_body, 0, unroll=4)

        def do_super(s, slot, gsem, wsem):
            row0 = s * _SUP

            @pl.when(s >= 2)
            def _():
                pltpu.make_async_copy(
                    rows_v.at[slot], out_hbm.at[pl.ds(base, _SUP)], wsem
                ).wait()

            descs = []
            for h in range(2):
                for j in range(_SUP_G):
                    descs.append(
                        pltpu.async_copy(
                            table_hbm.at[idx_v.at[h, pl.ds(row0 + j * _CHUNK, _CHUNK)]],
                            rows_v.at[slot, pl.ds(j * _CHUNK, _CHUNK), pl.ds(h * F, F)],
                            gsem,
                        )
                    )
            for d in descs:
                d.wait()

            pltpu.async_copy(
                rows_v.at[slot], out_hbm.at[pl.ds(base + row0, _SUP)], wsem
            )

        def body(p, carry):
            do_super(2 * p, 0, gs0, ws0)
            do_super(2 * p + 1, 1, gs1, ws1)
            return carry

        lax.fori_loop(0, n_sup // 2, body, 0)
        pltpu.make_async_copy(rows_v.at[0], out_hbm.at[pl.ds(base, _SUP)], ws0).wait()
        pltpu.make_async_copy(rows_v.at[1], out_hbm.at[pl.ds(base, _SUP)], ws1).wait()

    return gather_kernel(embed, idx)


def _tc_film_paired(e2, x2, w_pack, bg, bb, blk2):
    H, F2 = e2.shape
    D = x2.shape[1]
    nb = H // blk2

    def body(e_ref, xlo_ref, xhi_ref, w_ref, bg_ref, bb_ref, o_ref):
        gb4 = jnp.dot(e_ref[...], w_ref[...], preferred_element_type=jnp.float32)
        g_lo = gb4[:, :D]
        g_hi = gb4[:, D:2 * D]
        b_lo = gb4[:, 2 * D:3 * D]
        b_hi = gb4[:, 3 * D:]
        one_bg = 1.0 + bg_ref[...]
        o_ref[0] = (g_lo + one_bg) * xlo_ref[...] + (b_lo + bb_ref[...])
        o_ref[1] = (g_hi + one_bg) * xhi_ref[...] + (b_hi + bb_ref[...])

    return pl.pallas_call(
        body,
        grid=(nb,),
        in_specs=[
            pl.BlockSpec((blk2, F2), lambda i: (i, 0)),
            pl.BlockSpec((blk2, D), lambda i: (i, 0)),
            pl.BlockSpec((blk2, D), lambda i, _nb=nb: (i + _nb, 0)),
            pl.BlockSpec((F2, 4 * D), lambda i: (0, 0)),
            pl.BlockSpec((1, D), lambda i: (0, 0)),
            pl.BlockSpec((1, D), lambda i: (0, 0)),
        ],
        out_specs=pl.BlockSpec((2, blk2, D), lambda i: (0, i, 0)),
        out_shape=jax.ShapeDtypeStruct((2, H, D), jnp.float32),
    )(e2, x2, x2, w_pack, bg, bb)


def kernel(x, mods, embed, W_gamma, b_gamma, W_beta, b_beta):
    B, N, D = x.shape
    F = embed.shape[1]
    M = B * N
    idx = mods.reshape(M).astype(jnp.int32)
    e2 = _sc_gather_paired(embed, idx)
    zf = jnp.zeros((F, D), jnp.float32)
    w_pack = jnp.block([
        [W_gamma.T, zf, W_beta.T, zf],
        [zf, W_gamma.T, zf, W_beta.T],
    ])
    out2 = _tc_film_paired(
        e2,
        x.reshape(M, D),
        w_pack,
        b_gamma.reshape(1, D),
        b_beta.reshape(1, D),
        blk2=1024,
    )
    return out2.reshape(B, N, D)


# speedup vs baseline: 1.4910x; 1.4910x over previous
import functools

import jax
import jax.numpy as jnp
from jax import lax
from jax.experimental import pallas as pl
from jax.experimental.pallas import tpu as pltpu
from jax.experimental.pallas import tpu_sc as plsc

_NUM_CORES = 2
_NUM_SUBCORES = 16
_NW = _NUM_CORES * _NUM_SUBCORES
_LANES = 16

_CHUNK = 128
_SUP_G = 5
_SUP = _SUP_G * _CHUNK

_PAIR = 1600


def _sc_gather_paired(embed, idx):
    M = idx.shape[0]
    F = embed.shape[1]
    per_w = M // _NW
    n_sup = per_w // _SUP
    n_blocks = per_w // (2 * _PAIR)
    mesh = plsc.VectorSubcoreMesh(core_axis_name="c", subcore_axis_name="s")

    @functools.partial(
        pl.kernel,
        mesh=mesh,
        out_type=jax.ShapeDtypeStruct((M, F), jnp.float32),
        compiler_params=pltpu.CompilerParams(
            use_tc_tiling_on_sc=False, needs_layout_passes=False),
        scratch_types=[
            pltpu.VMEM((per_w,), jnp.int32),
            pltpu.VMEM((per_w,), jnp.int32),
            pltpu.VMEM((2, _SUP, F), jnp.float32),
            pltpu.SemaphoreType.DMA,
            pltpu.SemaphoreType.DMA,
            pltpu.SemaphoreType.DMA,
            pltpu.SemaphoreType.DMA,
        ],
    )
    def gather_kernel(table_hbm, idx_hbm, out_hbm, idx_v, int_v, rows_v,
                      gs0, gs1, ws0, ws1):
        wid = lax.axis_index("s") * _NUM_CORES + lax.axis_index("c")
        base = wid * per_w
        pltpu.sync_copy(idx_hbm.at[pl.ds(base, per_w)], idx_v)

        lane_pos = 2 * lax.iota(jnp.int32, _LANES)

        def ilv_body(i, carry):
            t = i // (_PAIR // _LANES)
            k = i % (_PAIR // _LANES)
            lo_off = t * 2 * _PAIR + k * _LANES
            lo = jnp.maximum(idx_v[pl.ds(lo_off, _LANES)], 1)
            hi = jnp.maximum(idx_v[pl.ds(lo_off + _PAIR, _LANES)], 1)
            pos = lane_pos + (t * 2 * _PAIR + 2 * k * _LANES)
            plsc.store_scatter(int_v, [pos], lo)
            plsc.store_scatter(int_v, [pos + 1], hi)
            return carry

        lax.fori_loop(0, n_blocks * (_PAIR // _LANES), ilv_body, 0, unroll=2)

        def do_super(s, slot, gsem, wsem):
            row0 = s * _SUP

            @pl.when(s >= 2)
            def _():
                pltpu.make_async_copy(
                    rows_v.at[slot], out_hbm.at[pl.ds(base, _SUP)], wsem
                ).wait()

            descs = [
                pltpu.async_copy(
                    table_hbm.at[int_v.at[pl.ds(row0 + j * _CHUNK, _CHUNK)]],
                    rows_v.at[slot, pl.ds(j * _CHUNK, _CHUNK)],
                    gsem,
                )
                for j in range(_SUP_G)
            ]
            for d in descs:
                d.wait()

            pltpu.async_copy(
                rows_v.at[slot], out_hbm.at[pl.ds(base + row0, _SUP)], wsem
            )

        def body(p, carry):
            do_super(2 * p, 0, gs0, ws0)
            do_super(2 * p + 1, 1, gs1, ws1)
            return carry

        lax.fori_loop(0, n_sup // 2, body, 0)
        pltpu.make_async_copy(rows_v.at[0], out_hbm.at[pl.ds(base, _SUP)], ws0).wait()
        pltpu.make_async_copy(rows_v.at[1], out_hbm.at[pl.ds(base, _SUP)], ws1).wait()

    return gather_kernel(embed, idx)


def _tc_film_paired(e2, x2, w_pack, bg, bb):
    H, F2 = e2.shape
    M, D = x2.shape
    blk2 = _PAIR
    blk = 2 * blk2
    nb = H // blk2

    def body(e_ref, x_ref, w_ref, bg_ref, bb_ref, o_ref):
        e_bf = e_ref[...].astype(jnp.bfloat16)
        gb4 = jnp.dot(e_bf, w_ref[...], preferred_element_type=jnp.float32)
        one_bg = 1.0 + bg_ref[...]
        o_ref[:blk2] = (gb4[:, :D] + one_bg) * x_ref[:blk2] + (
            gb4[:, 2 * D:3 * D] + bb_ref[...])
        o_ref[blk2:] = (gb4[:, D:2 * D] + one_bg) * x_ref[blk2:] + (
            gb4[:, 3 * D:] + bb_ref[...])

    return pl.pallas_call(
        body,
        grid=(nb,),
        in_specs=[
            pl.BlockSpec((blk2, F2), lambda i: (i, 0)),
            pl.BlockSpec((blk, D), lambda i: (i, 0)),
            pl.BlockSpec((F2, 4 * D), lambda i: (0, 0)),
            pl.BlockSpec((1, D), lambda i: (0, 0)),
            pl.BlockSpec((1, D), lambda i: (0, 0)),
        ],
        out_specs=pl.BlockSpec((blk, D), lambda i: (i, 0)),
        out_shape=jax.ShapeDtypeStruct((M, D), jnp.float32),
    )(e2, x2, w_pack, bg, bb)


def kernel(x, mods, embed, W_gamma, b_gamma, W_beta, b_beta):
    B, N, D = x.shape
    F = embed.shape[1]
    M = B * N
    idx = mods.reshape(M).astype(jnp.int32)
    e = _sc_gather_paired(embed, idx)
    e2 = e.reshape(M // 2, 2 * F)
    zf = jnp.zeros((F, D), jnp.float32)
    w_pack = jnp.block([
        [W_gamma.T, zf, W_beta.T, zf],
        [zf, W_gamma.T, zf, W_beta.T],
    ]).astype(jnp.bfloat16)
    out = _tc_film_paired(
        e2,
        x.reshape(M, D),
        w_pack,
        b_gamma.reshape(1, D),
        b_beta.reshape(1, D),
    )
    return out.reshape(B, N, D)
